# baseline (speedup 1.0000x reference)
import functools

import ml_dtypes
import numpy as np

import jax
import jax.numpy as jnp
from jax import lax
from jax.experimental import pallas as pl
from jax.experimental.pallas import tpu as pltpu
from jax.experimental.pallas import tpu_sc as plsc

N = 10000
E = 320000
D_IN = 128
D_H = 64
D_OUT = 64
D_AUG = 96

NC, NS = 2, 16
N_PAD = 10240
EPT = 20480
E_PAD = NS * EPT
BIGCHUNK = 512
NBIG = EPT // BIGCHUNK
ROWS_T = N_PAD // NS

_sc_mesh = plsc.VectorSubcoreMesh(
    core_axis_name="c", subcore_axis_name="s", num_cores=NC, num_subcores=NS)
_sc_params = pltpu.CompilerParams(use_tc_tiling_on_sc=False)


def _fill_1d(ref, val, n):
  def body(i, _):
    ref[pl.ds(i * 16, 16)] = jnp.full((16,), val, jnp.float32)
    return 0
  lax.fori_loop(0, n // 16, body, 0, unroll=False)


@functools.partial(
    pl.kernel,
    out_type=(jax.ShapeDtypeStruct((N_PAD,), jnp.float32),
              jax.ShapeDtypeStruct((N_PAD,), jnp.float32)),
    mesh=_sc_mesh,
    scratch_types=(
        pltpu.VMEM((NBIG, BIGCHUNK), jnp.int32),
        pltpu.VMEM((BIGCHUNK,), jnp.float32),
        pltpu.VMEM((ROWS_T,), jnp.float32),
        pltpu.VMEM_SHARED((N_PAD,), jnp.float32),
        pltpu.SemaphoreType.DMA,
    ),
    compiler_params=_sc_params,
)
def _sc_deg(dst_td, dst_bu, deg_td, deg_bu, idx_v, ones_v, zero1_v, sdeg, sem):
  c = lax.axis_index("c")
  s = lax.axis_index("s")
  _fill_1d(ones_v, 1.0, BIGCHUNK)
  _fill_1d(zero1_v, 0.0, ROWS_T)
  pltpu.sync_copy(zero1_v, sdeg.at[pl.ds(s * ROWS_T, ROWS_T)])

  @pl.when(c == 0)
  def _():
    pltpu.sync_copy(dst_td.at[s], idx_v)

  @pl.when(c == 1)
  def _():
    pltpu.sync_copy(dst_bu.at[s], idx_v)

  plsc.subcore_barrier()

  def block(b, _):
    for k in range(8):
      pltpu.async_copy(ones_v, sdeg.at[idx_v.at[b * 8 + k]], sem, add=True)
    for k in range(8):
      pltpu.make_async_copy(ones_v, sdeg.at[idx_v.at[b * 8 + k]], sem).wait()
    return 0
  lax.fori_loop(0, NBIG // 8, block, 0, unroll=False)
  plsc.subcore_barrier()

  @pl.when(c == 0)
  def _():
    pltpu.sync_copy(sdeg.at[pl.ds(s * ROWS_T, ROWS_T)],
                    deg_td.at[pl.ds(s * ROWS_T, ROWS_T)])

  @pl.when(c == 1)
  def _():
    pltpu.sync_copy(sdeg.at[pl.ds(s * ROWS_T, ROWS_T)],
                    deg_bu.at[pl.ds(s * ROWS_T, ROWS_T)])


def _make_edge_pass(width):
  dt = jnp.bfloat16
  out_type = (jax.ShapeDtypeStruct((N_PAD, width), dt),
              jax.ShapeDtypeStruct((N_PAD, width), dt))
  scratch = (
      pltpu.VMEM((NBIG, BIGCHUNK), jnp.int32),
      pltpu.VMEM((NBIG, BIGCHUNK), jnp.int32),
      pltpu.VMEM((BIGCHUNK, width), dt),
      pltpu.VMEM((BIGCHUNK, width), dt),
      pltpu.VMEM_SHARED((N_PAD, width), dt),
      pltpu.SemaphoreType.DMA,
      pltpu.SemaphoreType.DMA,
  )

  def body(z_td, z_bu, src_td, dst_td, src_bu, dst_bu, zrows,
           out_td, out_bu, src_v, dst_v, buf0, buf1, sacc,
           sem0, sem1):
    c = lax.axis_index("c")
    s = lax.axis_index("s")

    def idx(v, j):
      return v.at[j]

    def run(z_hbm, src_hbm, dst_hbm, s_out):
      pltpu.sync_copy(zrows, sacc.at[pl.ds(s * ROWS_T, ROWS_T)])
      pltpu.sync_copy(src_hbm.at[s], src_v)
      pltpu.sync_copy(dst_hbm.at[s], dst_v)
      plsc.subcore_barrier()

      pltpu.async_copy(z_hbm.at[idx(src_v, 0)], buf0, sem0)

      def step(j, buf, sem, nbuf, nsem):
        @pl.when(j + 1 < NBIG)
        def _():
          pltpu.async_copy(z_hbm.at[idx(src_v, j + 1)], nbuf, nsem)
        pltpu.make_async_copy(z_hbm.at[idx(src_v, j)], buf, sem).wait()
        pltpu.sync_copy(buf, sacc.at[idx(dst_v, j)], add=True)

      def pair(i, _):
        step(2 * i, buf0, sem0, buf1, sem1)
        step(2 * i + 1, buf1, sem1, buf0, sem0)
        return 0
      lax.fori_loop(0, NBIG // 2, pair, 0, unroll=False)
      plsc.subcore_barrier()
      pltpu.sync_copy(sacc.at[pl.ds(s * ROWS_T, ROWS_T)],
                      s_out.at[pl.ds(s * ROWS_T, ROWS_T)])

    @pl.when(c == 0)
    def _():
      run(z_td, src_td, dst_td, out_td)

    @pl.when(c == 1)
    def _():
      run(z_bu, src_bu, dst_bu, out_bu)

  return pl.kernel(body, out_type=out_type, mesh=_sc_mesh,
                   scratch_types=scratch, compiler_params=_sc_params)


_sc_pass1 = _make_edge_pass(D_AUG)
_sc_pass2 = _make_edge_pass(D_H)


RB = 2000


def _tc_pre_body(x_ref, degtd_ref, degbu_ref, w1td_ref, w1bu_ref,
                 z1td_ref, z1bu_ref):
  x = x_ref[...]

  def one(degcol, w1, out_ref):
    dinv = lax.rsqrt(degcol + 1.0)
    hi = dinv.astype(jnp.bfloat16)
    lo = (dinv - hi.astype(jnp.float32)).astype(jnp.bfloat16)
    z1 = jnp.dot(x, w1, preferred_element_type=jnp.float32) * dinv
    zeros = jnp.zeros((RB, D_AUG - D_H - 2), jnp.bfloat16)
    out_ref[...] = jnp.concatenate(
        [z1.astype(jnp.bfloat16), hi, lo, zeros], axis=1)

  one(degtd_ref[...], w1td_ref[...], z1td_ref)
  one(degbu_ref[...], w1bu_ref[...], z1bu_ref)


_tc_pre = pl.pallas_call(
    _tc_pre_body,
    grid=(N // RB,),
    in_specs=[
        pl.BlockSpec((RB, D_IN), lambda i: (i, 0)),
        pl.BlockSpec((RB, 1), lambda i: (i, 0)),
        pl.BlockSpec((RB, 1), lambda i: (i, 0)),
        pl.BlockSpec((D_IN, D_H), lambda i: (0, 0)),
        pl.BlockSpec((D_IN, D_H), lambda i: (0, 0)),
    ],
    out_specs=[
        pl.BlockSpec((RB, D_AUG), lambda i: (i, 0)),
        pl.BlockSpec((RB, D_AUG), lambda i: (i, 0)),
    ],
    out_shape=[
        jax.ShapeDtypeStruct((N_PAD, D_AUG), jnp.bfloat16),
        jax.ShapeDtypeStruct((N_PAD, D_AUG), jnp.bfloat16),
    ],
)


def _tc_mid_body(s1td_ref, s1bu_ref, z1td_ref, z1bu_ref,
                 w2atd_ref, w2abu_ref, rootx_ref, w2btd_ref, w2bbu_ref,
                 z2td_ref, z2bu_ref, ctd_ref, cbu_ref):
  def one(s1aug, z1aug, w2a, w2b):
    s1aug = s1aug.astype(jnp.float32)
    z1aug = z1aug.astype(jnp.float32)
    dinv = z1aug[:, D_H:D_H + 1] + z1aug[:, D_H + 1:D_H + 2]
    sd = s1aug[:, D_H:D_H + 1] + s1aug[:, D_H + 1:D_H + 2]
    h1 = jnp.maximum(dinv * (s1aug[:, :D_H] + z1aug[:, :D_H]), 0.0)
    z2 = jnp.dot(h1, w2a, preferred_element_type=jnp.float32) * dinv
    rw = jnp.dot(rootx_ref[...], w2b, preferred_element_type=jnp.float32)
    cc = dinv * z2 + (dinv * (sd + dinv)) * rw
    return z2.astype(jnp.bfloat16), cc

  z2td, ctd = one(s1td_ref[...], z1td_ref[...], w2atd_ref[...], w2btd_ref[...])
  z2bu, cbu = one(s1bu_ref[...], z1bu_ref[...], w2abu_ref[...], w2bbu_ref[...])
  z2td_ref[...] = z2td
  z2bu_ref[...] = z2bu
  ctd_ref[...] = ctd
  cbu_ref[...] = cbu


_tc_mid = pl.pallas_call(
    _tc_mid_body,
    grid=(N // RB,),
    in_specs=[
        pl.BlockSpec((RB, D_AUG), lambda i: (i, 0)),
        pl.BlockSpec((RB, D_AUG), lambda i: (i, 0)),
        pl.BlockSpec((RB, D_AUG), lambda i: (i, 0)),
        pl.BlockSpec((RB, D_AUG), lambda i: (i, 0)),
        pl.BlockSpec((D_H, D_OUT), lambda i: (0, 0)),
        pl.BlockSpec((D_H, D_OUT), lambda i: (0, 0)),
        pl.BlockSpec((1, D_IN), lambda i: (0, 0)),
        pl.BlockSpec((D_IN, D_OUT), lambda i: (0, 0)),
        pl.BlockSpec((D_IN, D_OUT), lambda i: (0, 0)),
    ],
    out_specs=[
        pl.BlockSpec((RB, D_OUT), lambda i: (i, 0)),
        pl.BlockSpec((RB, D_OUT), lambda i: (i, 0)),
        pl.BlockSpec((RB, D_OUT), lambda i: (i, 0)),
        pl.BlockSpec((RB, D_OUT), lambda i: (i, 0)),
    ],
    out_shape=[
        jax.ShapeDtypeStruct((N, D_OUT), jnp.bfloat16),
        jax.ShapeDtypeStruct((N, D_OUT), jnp.bfloat16),
        jax.ShapeDtypeStruct((N, D_OUT), jnp.float32),
        jax.ShapeDtypeStruct((N, D_OUT), jnp.float32),
    ],
)


def _tc_post_body(s2td_ref, s2bu_ref, ctd_ref, cbu_ref, z1td_ref, z1bu_ref,
                  out_ref):
  def one(s2, c, z1aug):
    dinv = (z1aug[:, D_H:D_H + 1].astype(jnp.float32)
            + z1aug[:, D_H + 1:D_H + 2].astype(jnp.float32))
    return jnp.maximum(dinv * s2.astype(jnp.float32) + c, 0.0)

  td = one(s2td_ref[...], ctd_ref[...], z1td_ref[...])
  bu = one(s2bu_ref[...], cbu_ref[...], z1bu_ref[...])
  out_ref[...] = jnp.concatenate([td, bu], axis=1)


_tc_post = pl.pallas_call(
    _tc_post_body,
    grid=(N // RB,),
    in_specs=[
        pl.BlockSpec((RB, D_OUT), lambda i: (i, 0)),
        pl.BlockSpec((RB, D_OUT), lambda i: (i, 0)),
        pl.BlockSpec((RB, D_OUT), lambda i: (i, 0)),
        pl.BlockSpec((RB, D_OUT), lambda i: (i, 0)),
        pl.BlockSpec((RB, D_AUG), lambda i: (i, 0)),
        pl.BlockSpec((RB, D_AUG), lambda i: (i, 0)),
    ],
    out_specs=pl.BlockSpec((RB, 2 * D_OUT), lambda i: (i, 0)),
    out_shape=jax.ShapeDtypeStruct((N, 2 * D_OUT), jnp.float32),
)


_Z1ROWS = np.zeros((ROWS_T, D_AUG), dtype=ml_dtypes.bfloat16)
_Z2ROWS = np.zeros((ROWS_T, D_H), dtype=ml_dtypes.bfloat16)


def _pad_edges(ei):
  pad = N + (jnp.arange(E_PAD - E, dtype=jnp.int32) % (N_PAD - N))
  src = jnp.concatenate([ei[0], pad]).reshape(NS, NBIG, BIGCHUNK)
  dst = jnp.concatenate([ei[1], pad]).reshape(NS, NBIG, BIGCHUNK)
  return src, dst


def kernel(x, edge_index, BU_edge_index, rootindex, W1_td, W2_td, W1_bu, W2_bu):
  src_td, dst_td = _pad_edges(edge_index)
  src_bu, dst_bu = _pad_edges(BU_edge_index)
  deg_td, deg_bu = _sc_deg(dst_td, dst_bu)

  z1_td, z1_bu = _tc_pre(x, deg_td[:N, None], deg_bu[:N, None], W1_td, W1_bu)

  s1_td, s1_bu = _sc_pass1(z1_td, z1_bu, src_td, dst_td, src_bu, dst_bu,
                           jnp.asarray(_Z1ROWS))

  root_x = lax.dynamic_slice_in_dim(x, rootindex[0], 1, axis=0)
  z2_td, z2_bu, c_td, c_bu = _tc_mid(
      s1_td, s1_bu, z1_td, z1_bu,
      W2_td[:D_H], W2_bu[:D_H], root_x, W2_td[D_H:], W2_bu[D_H:])

  s2_td, s2_bu = _sc_pass2(z2_td, z2_bu, src_td, dst_td, src_bu, dst_bu,
                           jnp.asarray(_Z2ROWS))

  return _tc_post(s2_td, s2_bu, c_td, c_bu, z1_td, z1_bu)


# speedup vs baseline: 1.0214x; 1.0214x over previous
import functools

import ml_dtypes
import numpy as np

import jax
import jax.numpy as jnp
from jax import lax
from jax.experimental import pallas as pl
from jax.experimental.pallas import tpu as pltpu
from jax.experimental.pallas import tpu_sc as plsc

N = 10000
E = 320000
D_IN = 128
D_H = 64
D_OUT = 64
D_AUG = 80

NC, NS = 2, 16
N_PAD = 10240
EPT = 20480
E_PAD = NS * EPT
BIGCHUNK = 512
NBIG = EPT // BIGCHUNK
ROWS_T = N_PAD // NS

_sc_mesh = plsc.VectorSubcoreMesh(
    core_axis_name="c", subcore_axis_name="s", num_cores=NC, num_subcores=NS)
_sc_params = pltpu.CompilerParams(use_tc_tiling_on_sc=False)


def _fill_1d(ref, val, n):
  def body(i, _):
    ref[pl.ds(i * 16, 16)] = jnp.full((16,), val, jnp.float32)
    return 0
  lax.fori_loop(0, n // 16, body, 0, unroll=False)


@functools.partial(
    pl.kernel,
    out_type=(jax.ShapeDtypeStruct((N_PAD,), jnp.float32),
              jax.ShapeDtypeStruct((N_PAD,), jnp.float32)),
    mesh=_sc_mesh,
    scratch_types=(
        pltpu.VMEM((NBIG, BIGCHUNK), jnp.int32),
        pltpu.VMEM((BIGCHUNK,), jnp.float32),
        pltpu.VMEM((ROWS_T,), jnp.float32),
        pltpu.VMEM_SHARED((N_PAD,), jnp.float32),
        pltpu.SemaphoreType.DMA,
    ),
    compiler_params=_sc_params,
)
def _sc_deg(dst_td, dst_bu, deg_td, deg_bu, idx_v, ones_v, zero1_v, sdeg, sem):
  c = lax.axis_index("c")
  s = lax.axis_index("s")
  _fill_1d(ones_v, 1.0, BIGCHUNK)
  _fill_1d(zero1_v, 0.0, ROWS_T)
  pltpu.sync_copy(zero1_v, sdeg.at[pl.ds(s * ROWS_T, ROWS_T)])

  @pl.when(c == 0)
  def _():
    pltpu.sync_copy(dst_td.at[s], idx_v)

  @pl.when(c == 1)
  def _():
    pltpu.sync_copy(dst_bu.at[s], idx_v)

  plsc.subcore_barrier()

  def block(b, _):
    for k in range(8):
      pltpu.async_copy(ones_v, sdeg.at[idx_v.at[b * 8 + k]], sem, add=True)
    for k in range(8):
      pltpu.make_async_copy(ones_v, sdeg.at[idx_v.at[b * 8 + k]], sem).wait()
    return 0
  lax.fori_loop(0, NBIG // 8, block, 0, unroll=False)
  plsc.subcore_barrier()

  @pl.when(c == 0)
  def _():
    pltpu.sync_copy(sdeg.at[pl.ds(s * ROWS_T, ROWS_T)],
                    deg_td.at[pl.ds(s * ROWS_T, ROWS_T)])

  @pl.when(c == 1)
  def _():
    pltpu.sync_copy(sdeg.at[pl.ds(s * ROWS_T, ROWS_T)],
                    deg_bu.at[pl.ds(s * ROWS_T, ROWS_T)])


def _make_edge_pass(width):
  dt = jnp.bfloat16
  out_type = (jax.ShapeDtypeStruct((N_PAD, width), dt),
              jax.ShapeDtypeStruct((N_PAD, width), dt))
  scratch = (
      pltpu.VMEM((NBIG, BIGCHUNK), jnp.int32),
      pltpu.VMEM((NBIG, BIGCHUNK), jnp.int32),
      pltpu.VMEM((BIGCHUNK, width), dt),
      pltpu.VMEM((BIGCHUNK, width), dt),
      pltpu.VMEM_SHARED((N_PAD, width), dt),
      pltpu.SemaphoreType.DMA,
      pltpu.SemaphoreType.DMA,
  )

  def body(z_td, z_bu, src_td, dst_td, src_bu, dst_bu, zrows,
           out_td, out_bu, src_v, dst_v, buf0, buf1, sacc,
           sem0, sem1):
    c = lax.axis_index("c")
    s = lax.axis_index("s")

    def idx(v, j):
      return v.at[j]

    def run(z_hbm, src_hbm, dst_hbm, s_out):
      pltpu.sync_copy(zrows, sacc.at[pl.ds(s * ROWS_T, ROWS_T)])
      pltpu.sync_copy(src_hbm.at[s], src_v)
      pltpu.sync_copy(dst_hbm.at[s], dst_v)
      plsc.subcore_barrier()

      pltpu.async_copy(z_hbm.at[idx(src_v, 0)], buf0, sem0)

      def step(j, buf, sem, nbuf, nsem):
        @pl.when(j + 1 < NBIG)
        def _():
          pltpu.async_copy(z_hbm.at[idx(src_v, j + 1)], nbuf, nsem)
        pltpu.make_async_copy(z_hbm.at[idx(src_v, j)], buf, sem).wait()
        pltpu.sync_copy(buf, sacc.at[idx(dst_v, j)], add=True)

      def pair(i, _):
        step(2 * i, buf0, sem0, buf1, sem1)
        step(2 * i + 1, buf1, sem1, buf0, sem0)
        return 0
      lax.fori_loop(0, NBIG // 2, pair, 0, unroll=False)
      plsc.subcore_barrier()
      pltpu.sync_copy(sacc.at[pl.ds(s * ROWS_T, ROWS_T)],
                      s_out.at[pl.ds(s * ROWS_T, ROWS_T)])

    @pl.when(c == 0)
    def _():
      run(z_td, src_td, dst_td, out_td)

    @pl.when(c == 1)
    def _():
      run(z_bu, src_bu, dst_bu, out_bu)

  return pl.kernel(body, out_type=out_type, mesh=_sc_mesh,
                   scratch_types=scratch, compiler_params=_sc_params)


_sc_pass1 = _make_edge_pass(D_AUG)
_sc_pass2 = _make_edge_pass(D_H)


RB = 2000


def _tc_pre_body(x_ref, degtd_ref, degbu_ref, w1td_ref, w1bu_ref,
                 z1td_ref, z1bu_ref):
  x = x_ref[...]

  def one(degcol, w1, out_ref):
    dinv = lax.rsqrt(degcol + 1.0)
    hi = dinv.astype(jnp.bfloat16)
    lo = (dinv - hi.astype(jnp.float32)).astype(jnp.bfloat16)
    z1 = jnp.dot(x, w1, preferred_element_type=jnp.float32) * dinv
    zeros = jnp.zeros((RB, D_AUG - D_H - 2), jnp.bfloat16)
    out_ref[...] = jnp.concatenate(
        [z1.astype(jnp.bfloat16), hi, lo, zeros], axis=1)

  one(degtd_ref[...], w1td_ref[...], z1td_ref)
  one(degbu_ref[...], w1bu_ref[...], z1bu_ref)


_tc_pre = pl.pallas_call(
    _tc_pre_body,
    grid=(N // RB,),
    in_specs=[
        pl.BlockSpec((RB, D_IN), lambda i: (i, 0)),
        pl.BlockSpec((RB, 1), lambda i: (i, 0)),
        pl.BlockSpec((RB, 1), lambda i: (i, 0)),
        pl.BlockSpec((D_IN, D_H), lambda i: (0, 0)),
        pl.BlockSpec((D_IN, D_H), lambda i: (0, 0)),
    ],
    out_specs=[
        pl.BlockSpec((RB, D_AUG), lambda i: (i, 0)),
        pl.BlockSpec((RB, D_AUG), lambda i: (i, 0)),
    ],
    out_shape=[
        jax.ShapeDtypeStruct((N_PAD, D_AUG), jnp.bfloat16),
        jax.ShapeDtypeStruct((N_PAD, D_AUG), jnp.bfloat16),
    ],
)


def _tc_mid_body(s1td_ref, s1bu_ref, z1td_ref, z1bu_ref,
                 w2atd_ref, w2abu_ref, rootx_ref, w2btd_ref, w2bbu_ref,
                 z2td_ref, z2bu_ref, ctd_ref, cbu_ref):
  def one(s1aug, z1aug, w2a, w2b):
    s1aug = s1aug.astype(jnp.float32)
    z1aug = z1aug.astype(jnp.float32)
    dinv = z1aug[:, D_H:D_H + 1] + z1aug[:, D_H + 1:D_H + 2]
    sd = s1aug[:, D_H:D_H + 1] + s1aug[:, D_H + 1:D_H + 2]
    h1 = jnp.maximum(dinv * (s1aug[:, :D_H] + z1aug[:, :D_H]), 0.0)
    z2 = jnp.dot(h1, w2a, preferred_element_type=jnp.float32) * dinv
    rw = jnp.dot(rootx_ref[...], w2b, preferred_element_type=jnp.float32)
    cc = dinv * z2 + (dinv * (sd + dinv)) * rw
    return z2.astype(jnp.bfloat16), cc

  z2td, ctd = one(s1td_ref[...], z1td_ref[...], w2atd_ref[...], w2btd_ref[...])
  z2bu, cbu = one(s1bu_ref[...], z1bu_ref[...], w2abu_ref[...], w2bbu_ref[...])
  z2td_ref[...] = z2td
  z2bu_ref[...] = z2bu
  ctd_ref[...] = ctd
  cbu_ref[...] = cbu


_tc_mid = pl.pallas_call(
    _tc_mid_body,
    grid=(N // RB,),
    in_specs=[
        pl.BlockSpec((RB, D_AUG), lambda i: (i, 0)),
        pl.BlockSpec((RB, D_AUG), lambda i: (i, 0)),
        pl.BlockSpec((RB, D_AUG), lambda i: (i, 0)),
        pl.BlockSpec((RB, D_AUG), lambda i: (i, 0)),
        pl.BlockSpec((D_H, D_OUT), lambda i: (0, 0)),
        pl.BlockSpec((D_H, D_OUT), lambda i: (0, 0)),
        pl.BlockSpec((1, D_IN), lambda i: (0, 0)),
        pl.BlockSpec((D_IN, D_OUT), lambda i: (0, 0)),
        pl.BlockSpec((D_IN, D_OUT), lambda i: (0, 0)),
    ],
    out_specs=[
        pl.BlockSpec((RB, D_OUT), lambda i: (i, 0)),
        pl.BlockSpec((RB, D_OUT), lambda i: (i, 0)),
        pl.BlockSpec((RB, D_OUT), lambda i: (i, 0)),
        pl.BlockSpec((RB, D_OUT), lambda i: (i, 0)),
    ],
    out_shape=[
        jax.ShapeDtypeStruct((N, D_OUT), jnp.bfloat16),
        jax.ShapeDtypeStruct((N, D_OUT), jnp.bfloat16),
        jax.ShapeDtypeStruct((N, D_OUT), jnp.float32),
        jax.ShapeDtypeStruct((N, D_OUT), jnp.float32),
    ],
)


def _tc_post_body(s2td_ref, s2bu_ref, ctd_ref, cbu_ref, z1td_ref, z1bu_ref,
                  out_ref):
  def one(s2, c, z1aug):
    dinv = (z1aug[:, D_H:D_H + 1].astype(jnp.float32)
            + z1aug[:, D_H + 1:D_H + 2].astype(jnp.float32))
    return jnp.maximum(dinv * s2.astype(jnp.float32) + c, 0.0)

  td = one(s2td_ref[...], ctd_ref[...], z1td_ref[...])
  bu = one(s2bu_ref[...], cbu_ref[...], z1bu_ref[...])
  out_ref[...] = jnp.concatenate([td, bu], axis=1)


_tc_post = pl.pallas_call(
    _tc_post_body,
    grid=(N // RB,),
    in_specs=[
        pl.BlockSpec((RB, D_OUT), lambda i: (i, 0)),
        pl.BlockSpec((RB, D_OUT), lambda i: (i, 0)),
        pl.BlockSpec((RB, D_OUT), lambda i: (i, 0)),
        pl.BlockSpec((RB, D_OUT), lambda i: (i, 0)),
        pl.BlockSpec((RB, D_AUG), lambda i: (i, 0)),
        pl.BlockSpec((RB, D_AUG), lambda i: (i, 0)),
    ],
    out_specs=pl.BlockSpec((RB, 2 * D_OUT), lambda i: (i, 0)),
    out_shape=jax.ShapeDtypeStruct((N, 2 * D_OUT), jnp.float32),
)


_Z1ROWS = np.zeros((ROWS_T, D_AUG), dtype=ml_dtypes.bfloat16)
_Z2ROWS = np.zeros((ROWS_T, D_H), dtype=ml_dtypes.bfloat16)


def _pad_edges(ei):
  pad = N + (jnp.arange(E_PAD - E, dtype=jnp.int32) % (N_PAD - N))
  src = jnp.concatenate([ei[0], pad]).reshape(NS, NBIG, BIGCHUNK)
  dst = jnp.concatenate([ei[1], pad]).reshape(NS, NBIG, BIGCHUNK)
  return src, dst


def kernel(x, edge_index, BU_edge_index, rootindex, W1_td, W2_td, W1_bu, W2_bu):
  src_td, dst_td = _pad_edges(edge_index)
  src_bu, dst_bu = _pad_edges(BU_edge_index)
  deg_td, deg_bu = _sc_deg(dst_td, dst_bu)

  z1_td, z1_bu = _tc_pre(x, deg_td[:N, None], deg_bu[:N, None], W1_td, W1_bu)

  s1_td, s1_bu = _sc_pass1(z1_td, z1_bu, src_td, dst_td, src_bu, dst_bu,
                           jnp.asarray(_Z1ROWS))

  root_x = lax.dynamic_slice_in_dim(x, rootindex[0], 1, axis=0)
  z2_td, z2_bu, c_td, c_bu = _tc_mid(
      s1_td, s1_bu, z1_td, z1_bu,
      W2_td[:D_H], W2_bu[:D_H], root_x, W2_td[D_H:], W2_bu[D_H:])

  s2_td, s2_bu = _sc_pass2(z2_td, z2_bu, src_td, dst_td, src_bu, dst_bu,
                           jnp.asarray(_Z2ROWS))

  return _tc_post(s2_td, s2_bu, c_td, c_bu, z1_td, z1_bu)
